# baseline (speedup 1.0000x reference)
import math

import jax
import jax.numpy as jnp
from jax.experimental import pallas as pl
from jax.experimental.pallas import tpu as pltpu

HEADS = 16
NUM_BUCKETS = 32
MAX_DISTANCE = 128
QLEN = 2048
KLEN = 2048
TW = 4096
SUB = 8
GROUP = 128


def _bias_kernel(off_ref, embT_ref, out_ref, vs_ref):
    offset = off_ref[0]

    lane = jax.lax.broadcasted_iota(jnp.int32, (SUB, TW), 1)
    sub = jax.lax.broadcasted_iota(jnp.int32, (SUB, TW), 0)
    t = lane + (SUB - 1) - sub
    d = t - (QLEN - 1) - offset
    n = -d
    half = NUM_BUCKETS // 2
    ret = jnp.where(n < 0, half, 0)
    na = jnp.abs(n)
    max_exact = half // 2
    val_large = jnp.full_like(na, max_exact)
    for k in range(1, half - max_exact):
        thresh = max_exact * (MAX_DISTANCE / max_exact) ** (k / (half - max_exact))
        val_large = val_large + (na >= math.ceil(thresh)).astype(jnp.int32)
    bucket = ret + jnp.where(na < max_exact, na, val_large)

    Vs8 = jnp.zeros((SUB, TW), jnp.float32)
    for b in range(NUM_BUCKETS):
        Vs8 = Vs8 + jnp.where(bucket == b, embT_ref[0, 0:1, b : b + 1], 0.0)

    for a in range(GROUP // SUB):
        shift = SUB * (GROUP // SUB - 1 - a)
        vs_ref[SUB * a : SUB * (a + 1), :] = pltpu.roll(Vs8, (TW - shift) % TW, 1)

    for g in range(QLEN // GROUP):
        s = (QLEN - 1) - (GROUP - 1) - GROUP * g
        out_ref[0, GROUP * g : GROUP * (g + 1), :] = vs_ref[:, s : s + KLEN]


def kernel(qlen, klen, emb):
    offset = (jnp.asarray(klen) - jnp.asarray(qlen)).astype(jnp.int32)
    off = jnp.reshape(offset, (1,))
    embT = emb.T.reshape(HEADS, 1, NUM_BUCKETS)

    out = pl.pallas_call(
        _bias_kernel,
        grid=(HEADS,),
        in_specs=[
            pl.BlockSpec(memory_space=pltpu.SMEM),
            pl.BlockSpec((1, 1, NUM_BUCKETS), lambda h: (h, 0, 0)),
        ],
        out_specs=pl.BlockSpec((1, QLEN, KLEN), lambda h: (h, 0, 0)),
        out_shape=jax.ShapeDtypeStruct((HEADS, QLEN, KLEN), jnp.float32),
        scratch_shapes=[pltpu.VMEM((GROUP, TW), jnp.float32)],
    )(off, embT)
    return out
